# baseline (speedup 1.0000x reference)
import jax
import jax.numpy as jnp
from jax.experimental import pallas as pl
from jax.experimental.pallas import tpu as pltpu


def _make_kernel(bm, nbuf):
    def _fused_kernel(x_ref, w_ref, adj_ref, b_ref, out_ref,
                      xw_ref, bufs_ref, sems):
        m = pl.program_id(0)
        nblk = pl.num_programs(0)

        def _copy(i, slot):
            return pltpu.make_async_copy(
                adj_ref.at[pl.ds(i * bm, bm), :],
                bufs_ref.at[slot],
                sems.at[slot])

        @pl.when(m == 0)
        def _():
            for w in range(nbuf):
                _copy(w, w).start()
            xw_ref[...] = jnp.dot(x_ref[...], w_ref[...],
                                  preferred_element_type=jnp.float32)

        slot = jax.lax.rem(m, nbuf)
        _copy(m, slot).wait()
        acc = jnp.dot(bufs_ref[slot], xw_ref[...],
                      preferred_element_type=jnp.float32)
        out_ref[...] = jnp.maximum(acc + b_ref[...], 0.0)

        @pl.when(m + nbuf < nblk)
        def _():
            _copy(m + nbuf, slot).start()

    return _fused_kernel


def kernel(input, adj, W, b):
    x = input.reshape(input.shape[-2], input.shape[-1])
    n, d_in = x.shape
    d_out = W.shape[1]

    bm = min(400, n)
    nblk = n // bm
    nbuf = min(3, nblk)

    out = pl.pallas_call(
        _make_kernel(bm, nbuf),
        grid=(nblk,),
        in_specs=[
            pl.BlockSpec((n, d_in), lambda m: (0, 0)),
            pl.BlockSpec((d_in, d_out), lambda m: (0, 0)),
            pl.BlockSpec(memory_space=pl.ANY),
            pl.BlockSpec((1, d_out), lambda m: (0, 0)),
        ],
        out_specs=pl.BlockSpec((bm, d_out), lambda m: (m, 0)),
        out_shape=jax.ShapeDtypeStruct((n, d_out), jnp.float32),
        scratch_shapes=[
            pltpu.VMEM((n, d_out), jnp.float32),
            pltpu.VMEM((nbuf, bm, n), jnp.float32),
            pltpu.SemaphoreType.DMA((nbuf,)),
        ],
    )(x, W, adj, b.reshape(1, d_out))

    return out[None]


# speedup vs baseline: 1.0390x; 1.0390x over previous
import jax
import jax.numpy as jnp
from jax.experimental import pallas as pl
from jax.experimental.pallas import tpu as pltpu


def _fused_kernel(x_ref, w_ref, adj_ref, b_ref, out_ref, xw_ref):
    @pl.when(pl.program_id(0) == 0)
    def _():
        xw_ref[...] = jnp.dot(x_ref[...], w_ref[...],
                              preferred_element_type=jnp.float32)

    acc = jnp.dot(adj_ref[...], xw_ref[...],
                  preferred_element_type=jnp.float32)
    out_ref[...] = jnp.maximum(acc + b_ref[...], 0.0)


def kernel(input, adj, W, b):
    x = input.reshape(input.shape[-2], input.shape[-1])
    n, d_in = x.shape
    d_out = W.shape[1]

    bm = min(400, n)
    out = pl.pallas_call(
        _fused_kernel,
        grid=(n // bm,),
        in_specs=[
            pl.BlockSpec((n, d_in), lambda m: (0, 0)),
            pl.BlockSpec((d_in, d_out), lambda m: (0, 0)),
            pl.BlockSpec((bm, n), lambda m: (m, 0)),
            pl.BlockSpec((1, d_out), lambda m: (0, 0)),
        ],
        out_specs=pl.BlockSpec((bm, d_out), lambda m: (m, 0)),
        out_shape=jax.ShapeDtypeStruct((n, d_out), jnp.float32),
        scratch_shapes=[pltpu.VMEM((n, d_out), jnp.float32)],
    )(x, W, adj, b.reshape(1, d_out))

    return out[None]
